# baseline (speedup 1.0000x reference)
import functools
import math

import jax
import jax.numpy as jnp
from jax import lax
from jax.experimental import pallas as pl
from jax.experimental.pallas import tpu as pltpu
from jax.experimental.pallas import tpu_sc as plsc

NSRC = 10000
NDST = 10000
DEG = 32
D = 128
CCD = 128
NCODES = 32
K = 5
SIGMA = 1.0
E = NDST * DEG

NC = 2
NS = 16
NW = NC * NS
EPW = E // NW
CH = 80
NCH = EPW // CH
NBUF = 5
LPW = NSRC // 16

BB = 128
NB = (NDST + BB - 1) // BB

ISQRT_INDEG = float(1.0 / math.sqrt(float(DEG)))


def _wid():
    return lax.axis_index("s") * NC + lax.axis_index("c")


def _deg_body(src_hbm, part_hbm, idx_v, hist_v):
    base = _wid() * EPW
    pltpu.sync_copy(src_hbm.at[pl.ds(base, EPW)], idx_v)
    zf = jnp.zeros((16,), jnp.float32)

    def zero(i, c):
        hist_v[pl.ds(i * 16, 16)] = zf
        return c

    lax.fori_loop(0, LPW, zero, 0)
    onef = jnp.ones((16,), jnp.float32)

    def acc(i, c):
        idx = idx_v[pl.ds(i * 16, 16)]
        plsc.addupdate_scatter(hist_v, [idx], onef)
        return c

    lax.fori_loop(0, EPW // 16, acc, 0)
    pltpu.sync_copy(hist_v, part_hbm.at[_wid()])


def _gather_body(feat_hbm, src_hbm, mb_hbm, idx_v, buf_v, *sems):
    base = _wid() * EPW
    pltpu.sync_copy(src_hbm.at[pl.ds(base, EPW)], idx_v)

    def idx_slice(ci):
        return idx_v.at[pl.ds(pl.multiple_of(ci * CH, 8), CH)]

    for b in range(NBUF):
        pltpu.async_copy(feat_hbm.at[idx_slice(b)], buf_v.at[b], sems[b])

    def outer(oi, c):
        for b in range(NBUF):
            ci = oi * NBUF + b
            pltpu.make_async_copy(
                feat_hbm.at[idx_slice(ci)], buf_v.at[b], sems[b]).wait()
            pltpu.sync_copy(
                buf_v.at[b],
                mb_hbm.at[pl.ds(pl.multiple_of(base + ci * CH, 8), CH)])

            @pl.when(ci + NBUF < NCH)
            def _():
                pltpu.async_copy(
                    feat_hbm.at[idx_slice(ci + NBUF)], buf_v.at[b], sems[b])
        return c

    lax.fori_loop(0, NCH // NBUF, outer, 0)


def _sum_last(x):
    n = x.shape[-1]
    y = x[..., 0:8]
    for t in range(1, n // 8):
        y = y + x[..., 8 * t:8 * t + 8]
    y = y[..., 0:4] + y[..., 4:8]
    y = y[..., 0:2] + y[..., 2:4]
    return y[..., 0:1] + y[..., 1:2]


def _dist_block(m):
    G = lax.dot_general(m, m, (((2,), (2,)), ((0,), (0,))))
    sq = _sum_last(m * m)[..., 0]
    d2 = sq[:, :, None] + sq[:, None, :] - 2.0 * G
    return jnp.sqrt(jnp.clip(d2, 1e-12, None))


_sum32 = _sum_last


def _main_body(mb_ref, w_ref, cc_ref, muti_ref, rst_ref):
    m = mb_ref[...]
    dist = _dist_block(m)
    s1 = _sum32(dist)[:, :, 0] * (1.0 / DEG)
    meanv = (_sum32(s1) * (1.0 / DEG))[:, :, None]
    sims = jnp.exp(-dist / (SIGMA * meanv))

    cache = jnp.zeros((BB, 1, DEG), jnp.float32)
    li = lax.broadcasted_iota(jnp.int32, (BB, DEG), 1)
    mails = []
    for _ in range(K):
        gain = _sum32(jnp.maximum(sims, cache) - cache)[:, :, 0]
        mx = gain.max(axis=1, keepdims=True)
        cand = jnp.where(gain == mx, li, DEG)
        fm = cand.min(axis=1, keepdims=True)
        fst = (li == fm).astype(jnp.float32)
        row = (sims * fst[:, None, :]).sum(axis=2)
        cache = jnp.maximum(cache, row[:, None, :])
        mails.append((m * fst[:, :, None]).sum(axis=1))

    h = mails[0]
    for mk in mails[1:]:
        h = h + mk
    rst_ref[...] = h * ISQRT_INDEG

    w = w_ref[...]
    cc = cc_ref[...]
    wts = []
    for mk in mails:
        pj = jnp.tanh(lax.dot_general(mk, w, (((1,), (1,)), ((), ()))))
        wts.append(lax.dot_general(pj, cc, (((1,), (1,)), ((), ()))))

    mxw = wts[0]
    for t in wts[1:]:
        mxw = jnp.maximum(mxw, t)
    es = [jnp.exp(t - mxw) for t in wts]
    ssum = es[0]
    for t in es[1:]:
        ssum = ssum + t
    inv = 1.0 / ssum

    acc = jnp.zeros((BB, NCODES, D), jnp.float32)
    for e_, mk in zip(es, mails):
        p = e_ * inv
        acc = acc + p[:, :, None] * mk[:, None, :]
    muti_ref[...] = acc


def _sc_mesh():
    return plsc.VectorSubcoreMesh(core_axis_name="c", subcore_axis_name="s")


def _deg_partials(src):
    f = pl.kernel(
        _deg_body,
        mesh=_sc_mesh(),
        out_type=jax.ShapeDtypeStruct((NW, NSRC), jnp.float32),
        scratch_types=[
            pltpu.VMEM((EPW,), jnp.int32),
            pltpu.VMEM((NSRC,), jnp.float32),
        ],
        compiler_params=pltpu.CompilerParams(needs_layout_passes=False),
    )
    return f(src)


def _gather(feat, src):
    f = pl.kernel(
        _gather_body,
        mesh=_sc_mesh(),
        out_type=jax.ShapeDtypeStruct((E, D), jnp.float32),
        scratch_types=[
            pltpu.VMEM((EPW,), jnp.int32),
            pltpu.VMEM((NBUF, CH, D), jnp.float32),
        ] + [pltpu.SemaphoreType.DMA] * NBUF,
    )
    return f(feat, src)


def _main(mb3, W, cc):
    return pl.pallas_call(
        _main_body,
        grid=(NB,),
        in_specs=[
            pl.BlockSpec((BB, DEG, D), lambda i: (i, 0, 0)),
            pl.BlockSpec((CCD, D), lambda i: (0, 0)),
            pl.BlockSpec((NCODES, CCD), lambda i: (0, 0)),
        ],
        out_specs=[
            pl.BlockSpec((BB, NCODES, D), lambda i: (i, 0, 0)),
            pl.BlockSpec((BB, D), lambda i: (i, 0)),
        ],
        out_shape=[
            jax.ShapeDtypeStruct((NDST, NCODES, D), jnp.float32),
            jax.ShapeDtypeStruct((NDST, D), jnp.float32),
        ],
        compiler_params=pltpu.CompilerParams(
            dimension_semantics=("arbitrary",),
            vmem_limit_bytes=100 * 1024 * 1024,
        ),
    )(mb3, W, cc)


def kernel(x, edge_index, category, W, context_codes):
    del category
    src = edge_index[0]
    parts = _deg_partials(src)
    out_deg = jnp.clip(parts.sum(axis=0), 1.0, None)
    feat = x * (out_deg ** -0.5)[:, None]
    mailbox = _gather(feat, src)
    mb3 = mailbox.reshape(NDST, DEG, D)
    muti, rst = _main(mb3, W, context_codes)
    return (muti, rst)


# speedup vs baseline: 4.0421x; 4.0421x over previous
import functools
import math

import jax
import jax.numpy as jnp
from jax import lax
from jax.experimental import pallas as pl
from jax.experimental.pallas import tpu as pltpu
from jax.experimental.pallas import tpu_sc as plsc

NSRC = 10000
NDST = 10000
DEG = 32
D = 128
CCD = 128
NCODES = 32
K = 5
SIGMA = 1.0
E = NDST * DEG

NC = 2
NS = 16
NW = NC * NS
EPW = E // NW
CH = 80
NCH = EPW // CH
NBUF = 5
LPW = NSRC // 16

BB = 128
NB = (NDST + BB - 1) // BB

ISQRT_INDEG = float(1.0 / math.sqrt(float(DEG)))


def _wid():
    return lax.axis_index("s") * NC + lax.axis_index("c")


def _deg_body(src_hbm, part_hbm, idx_v, hist_v):
    base = _wid() * EPW
    pltpu.sync_copy(src_hbm.at[pl.ds(base, EPW)], idx_v)
    zf = jnp.zeros((16,), jnp.float32)

    def zero(i, c):
        hist_v[pl.ds(i * 16, 16)] = zf
        return c

    lax.fori_loop(0, LPW, zero, 0)
    onef = jnp.ones((16,), jnp.float32)

    def acc(i, c):
        idx = idx_v[pl.ds(i * 16, 16)]
        plsc.addupdate_scatter(hist_v, [idx], onef)
        return c

    lax.fori_loop(0, EPW // 16, acc, 0)
    pltpu.sync_copy(hist_v, part_hbm.at[_wid()])


def _gather_body(feat_hbm, src_hbm, mb_hbm, idx_v, buf_v, *sems):
    base = _wid() * EPW
    pltpu.sync_copy(src_hbm.at[pl.ds(base, EPW)], idx_v)

    def idx_slice(ci):
        return idx_v.at[pl.ds(pl.multiple_of(ci * CH, 8), CH)]

    for b in range(NBUF):
        pltpu.async_copy(feat_hbm.at[idx_slice(b)], buf_v.at[b], sems[b])

    def outer(oi, c):
        for b in range(NBUF):
            ci = oi * NBUF + b
            pltpu.make_async_copy(
                feat_hbm.at[idx_slice(ci)], buf_v.at[b], sems[b]).wait()
            pltpu.sync_copy(
                buf_v.at[b],
                mb_hbm.at[pl.ds(pl.multiple_of(base + ci * CH, 8), CH)])

            @pl.when(ci + NBUF < NCH)
            def _():
                pltpu.async_copy(
                    feat_hbm.at[idx_slice(ci + NBUF)], buf_v.at[b], sems[b])
        return c

    lax.fori_loop(0, NCH // NBUF, outer, 0)


def _subfold(x):
    n = x.shape[1]
    y = x[:, 0:8]
    for t in range(1, n // 8):
        y = y + x[:, 8 * t:8 * t + 8]
    y = y[:, 0:4] + y[:, 4:8]
    y = y[:, 0:2] + y[:, 2:4]
    return y[:, 0:1] + y[:, 1:2]


def _lanefold32(x):
    y = ((x[..., 0:8] + x[..., 8:16]) + x[..., 16:24]) + x[..., 24:32]
    y = y[..., 0:4] + y[..., 4:8]
    y = y[..., 0:2] + y[..., 2:4]
    return y[..., 0:1] + y[..., 1:2]


def _main_body(mb_ref, w_ref, cc_ref, muti_ref, rst_ref):
    m = mb_ref[...]
    G = lax.dot_general(m, m, (((2,), (2,)), ((0,), (0,))))
    pT = jnp.swapaxes(m * m, 1, 2)
    sqr = _subfold(pT)
    sqd = jnp.swapaxes(sqr, 1, 2)
    d2 = sqd + sqr - 2.0 * G
    dist = jnp.sqrt(jnp.clip(d2, 1e-12, None))
    s1 = _subfold(dist) * (1.0 / DEG)
    meanv = _lanefold32(s1) * (1.0 / DEG)
    sims = jnp.exp(-dist / (SIGMA * meanv))

    cache = jnp.zeros((BB, DEG, 1), jnp.float32)
    li = lax.broadcasted_iota(jnp.int32, (BB, 1, DEG), 2)
    fsts = []
    for _ in range(K):
        g = jnp.maximum(sims, cache) - cache
        gain = _subfold(g)
        mx = gain.max(axis=2, keepdims=True)
        cand = jnp.where(gain == mx, li, DEG)
        fm = cand.min(axis=2, keepdims=True)
        fst = (li == fm).astype(jnp.float32)
        row = (sims * fst).sum(axis=2, keepdims=True)
        cache = jnp.maximum(cache, row)
        fsts.append(fst)

    oh = jnp.concatenate(fsts, axis=1)
    mails = lax.dot_general(oh, m, (((2,), (1,)), ((0,), (0,))))
    rst_ref[...] = mails.sum(axis=1) * ISQRT_INDEG

    w = w_ref[...]
    cc = cc_ref[...]
    proj = jnp.tanh(lax.dot_general(mails, w, (((2,), (1,)), ((), ()))))
    wts = lax.dot_general(proj, cc, (((2,), (1,)), ((), ())))
    mxw = wts.max(axis=1, keepdims=True)
    es = jnp.exp(wts - mxw)
    p = es / es.sum(axis=1, keepdims=True)
    muti_ref[...] = lax.dot_general(p, mails, (((1,), (1,)), ((0,), (0,))))


def _sc_mesh():
    return plsc.VectorSubcoreMesh(core_axis_name="c", subcore_axis_name="s")


def _deg_partials(src):
    f = pl.kernel(
        _deg_body,
        mesh=_sc_mesh(),
        out_type=jax.ShapeDtypeStruct((NW, NSRC), jnp.float32),
        scratch_types=[
            pltpu.VMEM((EPW,), jnp.int32),
            pltpu.VMEM((NSRC,), jnp.float32),
        ],
        compiler_params=pltpu.CompilerParams(needs_layout_passes=False),
    )
    return f(src)


def _gather(feat, src):
    f = pl.kernel(
        _gather_body,
        mesh=_sc_mesh(),
        out_type=jax.ShapeDtypeStruct((E, D), jnp.float32),
        scratch_types=[
            pltpu.VMEM((EPW,), jnp.int32),
            pltpu.VMEM((NBUF, CH, D), jnp.float32),
        ] + [pltpu.SemaphoreType.DMA] * NBUF,
    )
    return f(feat, src)


def _main(mb3, W, cc):
    return pl.pallas_call(
        _main_body,
        grid=(NB,),
        in_specs=[
            pl.BlockSpec((BB, DEG, D), lambda i: (i, 0, 0)),
            pl.BlockSpec((CCD, D), lambda i: (0, 0)),
            pl.BlockSpec((NCODES, CCD), lambda i: (0, 0)),
        ],
        out_specs=[
            pl.BlockSpec((BB, NCODES, D), lambda i: (i, 0, 0)),
            pl.BlockSpec((BB, D), lambda i: (i, 0)),
        ],
        out_shape=[
            jax.ShapeDtypeStruct((NDST, NCODES, D), jnp.float32),
            jax.ShapeDtypeStruct((NDST, D), jnp.float32),
        ],
        compiler_params=pltpu.CompilerParams(
            dimension_semantics=("arbitrary",),
            vmem_limit_bytes=100 * 1024 * 1024,
        ),
    )(mb3, W, cc)


def kernel(x, edge_index, category, W, context_codes):
    del category
    src = edge_index[0]
    parts = _deg_partials(src)
    out_deg = jnp.clip(parts.sum(axis=0), 1.0, None)
    feat = x * (out_deg ** -0.5)[:, None]
    mailbox = _gather(feat, src)
    mb3 = mailbox.reshape(NDST, DEG, D)
    muti, rst = _main(mb3, W, context_codes)
    return (muti, rst)


# speedup vs baseline: 4.3108x; 1.0665x over previous
import functools
import math

import jax
import jax.numpy as jnp
from jax import lax
from jax.experimental import pallas as pl
from jax.experimental.pallas import tpu as pltpu
from jax.experimental.pallas import tpu_sc as plsc

NSRC = 10000
NDST = 10000
DEG = 32
D = 128
CCD = 128
NCODES = 32
K = 5
SIGMA = 1.0
E = NDST * DEG

NC = 2
NS = 16
NW = NC * NS
EPW = E // NW
CH = 80
NCH = EPW // CH
NBUF = 5
LPW = NSRC // 16

BB = 128
NB = (NDST + BB - 1) // BB

ISQRT_INDEG = float(1.0 / math.sqrt(float(DEG)))


def _wid():
    return lax.axis_index("s") * NC + lax.axis_index("c")


def _deg_body(src_hbm, part_hbm, idx_v, hist_v):
    base = _wid() * EPW
    pltpu.sync_copy(src_hbm.at[pl.ds(base, EPW)], idx_v)
    zf = jnp.zeros((16,), jnp.float32)

    def zero(i, c):
        hist_v[pl.ds(i * 16, 16)] = zf
        return c

    lax.fori_loop(0, LPW, zero, 0)
    onef = jnp.ones((16,), jnp.float32)

    def acc(i, c):
        idx = idx_v[pl.ds(i * 16, 16)]
        plsc.addupdate_scatter(hist_v, [idx], onef)
        return c

    lax.fori_loop(0, EPW // 16, acc, 0)
    pltpu.sync_copy(hist_v, part_hbm.at[_wid()])


def _gather_body(feat_hbm, src_hbm, mb_hbm, idx_v, buf_v, *sems):
    base = _wid() * EPW
    pltpu.sync_copy(src_hbm.at[pl.ds(base, EPW)], idx_v)

    def idx_slice(ci):
        return idx_v.at[pl.ds(pl.multiple_of(ci * CH, 8), CH)]

    for b in range(NBUF):
        pltpu.async_copy(feat_hbm.at[idx_slice(b)], buf_v.at[b], sems[b])

    def outer(oi, c):
        for b in range(NBUF):
            ci = oi * NBUF + b
            pltpu.make_async_copy(
                feat_hbm.at[idx_slice(ci)], buf_v.at[b], sems[b]).wait()
            pltpu.sync_copy(
                buf_v.at[b],
                mb_hbm.at[pl.ds(pl.multiple_of(base + ci * CH, 8), CH)])

            @pl.when(ci + NBUF < NCH)
            def _():
                pltpu.async_copy(
                    feat_hbm.at[idx_slice(ci + NBUF)], buf_v.at[b], sems[b])
        return c

    lax.fori_loop(0, NCH // NBUF, outer, 0)


def _subfold(x):
    n = x.shape[1]
    y = x[:, 0:8]
    for t in range(1, n // 8):
        y = y + x[:, 8 * t:8 * t + 8]
    y = y[:, 0:4] + y[:, 4:8]
    y = y[:, 0:2] + y[:, 2:4]
    return y[:, 0:1] + y[:, 1:2]


def _lanefold32(x):
    y = ((x[..., 0:8] + x[..., 8:16]) + x[..., 16:24]) + x[..., 24:32]
    y = y[..., 0:4] + y[..., 4:8]
    y = y[..., 0:2] + y[..., 2:4]
    return y[..., 0:1] + y[..., 1:2]


def _main_body(mb_ref, w_ref, cc_ref, muti_ref, rst_ref):
    m = mb_ref[...]
    G = lax.dot_general(m, m, (((2,), (2,)), ((0,), (0,))))
    pT = jnp.swapaxes(m * m, 1, 2)
    sqr = _subfold(pT)
    sqd = jnp.swapaxes(sqr, 1, 2)
    d2 = sqd + sqr - 2.0 * G
    dist = jnp.sqrt(jnp.clip(d2, 1e-12, None))
    s1 = _subfold(dist) * (1.0 / DEG)
    meanv = _lanefold32(s1) * (1.0 / DEG)
    sims = jnp.exp(-dist / (SIGMA * meanv))

    cache = jnp.zeros((BB, DEG, 1), jnp.float32)
    lif = lax.broadcasted_iota(jnp.int32, (BB, 1, DEG), 2).astype(jnp.float32)
    fsts = []
    for k in range(K):
        g = jnp.maximum(sims, cache) - cache
        gain = _subfold(g)
        mx = gain.max(axis=2, keepdims=True)
        cand = jnp.where(gain == mx, lif, float(DEG))
        fm = cand.min(axis=2, keepdims=True)
        fst = (lif == fm).astype(jnp.float32)
        fsts.append(fst)
        if k + 1 < K:
            row = (sims * fst).sum(axis=2, keepdims=True)
            cache = jnp.maximum(cache, row)

    oh = jnp.concatenate(fsts, axis=1)
    mails = lax.dot_general(oh, m, (((2,), (1,)), ((0,), (0,))))
    rst_ref[...] = mails.sum(axis=1) * ISQRT_INDEG

    w = w_ref[...]
    cc = cc_ref[...]
    proj = jnp.tanh(lax.dot_general(mails, w, (((2,), (1,)), ((), ()))))
    wts = lax.dot_general(proj, cc, (((2,), (1,)), ((), ())))
    mxw = wts.max(axis=1, keepdims=True)
    es = jnp.exp(wts - mxw)
    p = es * (1.0 / es.sum(axis=1, keepdims=True))
    muti_ref[...] = lax.dot_general(p, mails, (((1,), (1,)), ((0,), (0,))))


def _sc_mesh():
    return plsc.VectorSubcoreMesh(core_axis_name="c", subcore_axis_name="s")


def _deg_partials(src):
    f = pl.kernel(
        _deg_body,
        mesh=_sc_mesh(),
        out_type=jax.ShapeDtypeStruct((NW, NSRC), jnp.float32),
        scratch_types=[
            pltpu.VMEM((EPW,), jnp.int32),
            pltpu.VMEM((NSRC,), jnp.float32),
        ],
        compiler_params=pltpu.CompilerParams(needs_layout_passes=False),
    )
    return f(src)


def _gather(feat, src):
    f = pl.kernel(
        _gather_body,
        mesh=_sc_mesh(),
        out_type=jax.ShapeDtypeStruct((E, D), jnp.float32),
        scratch_types=[
            pltpu.VMEM((EPW,), jnp.int32),
            pltpu.VMEM((NBUF, CH, D), jnp.float32),
        ] + [pltpu.SemaphoreType.DMA] * NBUF,
    )
    return f(feat, src)


def _main(mb3, W, cc):
    return pl.pallas_call(
        _main_body,
        grid=(NB,),
        in_specs=[
            pl.BlockSpec((BB, DEG, D), lambda i: (i, 0, 0)),
            pl.BlockSpec((CCD, D), lambda i: (0, 0)),
            pl.BlockSpec((NCODES, CCD), lambda i: (0, 0)),
        ],
        out_specs=[
            pl.BlockSpec((BB, NCODES, D), lambda i: (i, 0, 0)),
            pl.BlockSpec((BB, D), lambda i: (i, 0)),
        ],
        out_shape=[
            jax.ShapeDtypeStruct((NDST, NCODES, D), jnp.float32),
            jax.ShapeDtypeStruct((NDST, D), jnp.float32),
        ],
        compiler_params=pltpu.CompilerParams(
            dimension_semantics=("arbitrary",),
            vmem_limit_bytes=100 * 1024 * 1024,
        ),
    )(mb3, W, cc)


def kernel(x, edge_index, category, W, context_codes):
    del category
    src = edge_index[0]
    parts = _deg_partials(src)
    out_deg = jnp.clip(parts.sum(axis=0), 1.0, None)
    feat = x * (out_deg ** -0.5)[:, None]
    mailbox = _gather(feat, src)
    mb3 = mailbox.reshape(NDST, DEG, D)
    muti, rst = _main(mb3, W, context_codes)
    return (muti, rst)
